# baseline (speedup 1.0000x reference)
import functools

import jax
import jax.numpy as jnp
from jax import lax
from jax.experimental import pallas as pl
from jax.experimental.pallas import tpu as pltpu
from jax.experimental.pallas import tpu_sc as plsc

_N = 2048
_E = 32768
_F = 128
_EPS = 1e-5

_NC = 2
_NS = 16
_NW = _NC * _NS

_SC_MESH = dict(core_axis_name="c", subcore_axis_name="s", num_cores=_NC,
                num_subcores=_NS)


def _sc_emb_body(table, idx, out, idx_v, rows_v, sem):
    wid = lax.axis_index("s") * _NC + lax.axis_index("c")
    rows = _N // _NW
    base = wid * rows
    pltpu.sync_copy(idx.at[pl.ds(base, rows)], idx_v)
    pltpu.async_copy(table.at[idx_v], rows_v, sem).wait()
    pltpu.sync_copy(rows_v, out.at[pl.ds(base, rows)])


@functools.cache
def _make_sc_emb():
    rows = _N // _NW
    return pl.kernel(
        _sc_emb_body,
        out_type=jax.ShapeDtypeStruct((_N, _F), jnp.float32),
        mesh=plsc.VectorSubcoreMesh(**_SC_MESH),
        scratch_types=[
            pltpu.VMEM((rows,), jnp.int32),
            pltpu.VMEM((rows, _F), jnp.float32),
            pltpu.SemaphoreType.DMA,
        ],
    )


_EPW = _E // _NW
_CH = 128
_NCH = _EPW // _CH


def _sc_edge_body(y1, y2, i1, i2, out, stats,
                  i1_v, i2_v, r1_0, r2_0, r1_1, r2_1, o_0, o_1, sv0, sv1,
                  sg1_0, sg2_0, sg1_1, sg2_1, so_0, so_1):
    wid = lax.axis_index("s") * _NC + lax.axis_index("c")
    base = wid * _EPW
    pltpu.sync_copy(i1.at[pl.ds(base, _EPW)], i1_v)
    pltpu.sync_copy(i2.at[pl.ds(base, _EPW)], i2_v)
    r1 = (r1_0, r1_1)
    r2 = (r2_0, r2_1)
    o = (o_0, o_1)
    sg1 = (sg1_0, sg1_1)
    sg2 = (sg2_0, sg2_1)
    so = (so_0, so_1)

    def fire(c):
        b = c & 1
        cp1 = pltpu.async_copy(y1.at[i1_v.at[pl.ds(c * _CH, _CH)]],
                               r1[b], sg1[b])
        cp2 = pltpu.async_copy(y2.at[i2_v.at[pl.ds(c * _CH, _CH)]],
                               r2[b], sg2[b])
        return cp1, cp2

    pending = {0: fire(0)}
    out_pending = {}
    carry = tuple(jnp.zeros((16,), jnp.float32) for _ in range(16))
    for c in range(_NCH):
        b = c & 1
        if c + 1 < _NCH:
            pending[c + 1] = fire(c + 1)
        cp1, cp2 = pending.pop(c)
        cp1.wait()
        cp2.wait()
        if c >= 2:
            out_pending.pop(c - 2).wait()
        r1v, r2v, ov = r1[b], r2[b], o[b]

        def row_body(it, carry):
            new = list(carry)
            for k in range(4):
                r = it * 4 + k
                for j in range(_F // 16):
                    sl = pl.ds(j * 16, 16)
                    a = r1v[r, sl] + r2v[r, sl]
                    ov[r, sl] = a
                    new[j] = new[j] + a
                    new[8 + j] = new[8 + j] + a * a
            return tuple(new)

        carry = lax.fori_loop(0, _CH // 4, row_body, carry)
        out_pending[c] = pltpu.async_copy(
            ov, out.at[pl.ds(base + c * _CH, _CH)], so[b])
    for c in sorted(out_pending):
        out_pending.pop(c).wait()
    for j in range(_F // 16):
        sl = pl.ds(j * 16, 16)
        sv0[0, sl] = carry[j]
        sv1[0, sl] = carry[8 + j]
    pltpu.sync_copy(sv0, stats.at[pl.ds(wid, 1)])
    pltpu.sync_copy(sv1, stats.at[pl.ds(_NW + wid, 1)])


@functools.cache
def _make_sc_edge():
    return pl.kernel(
        _sc_edge_body,
        out_type=[
            jax.ShapeDtypeStruct((_E, _F), jnp.float32),
            jax.ShapeDtypeStruct((2 * _NW, _F), jnp.float32),
        ],
        mesh=plsc.VectorSubcoreMesh(**_SC_MESH),
        scratch_types=[
            pltpu.VMEM((_EPW,), jnp.int32),
            pltpu.VMEM((_EPW,), jnp.int32),
            pltpu.VMEM((_CH, _F), jnp.float32),
            pltpu.VMEM((_CH, _F), jnp.float32),
            pltpu.VMEM((_CH, _F), jnp.float32),
            pltpu.VMEM((_CH, _F), jnp.float32),
            pltpu.VMEM((_CH, _F), jnp.float32),
            pltpu.VMEM((_CH, _F), jnp.float32),
            pltpu.VMEM((1, _F), jnp.float32),
            pltpu.VMEM((1, _F), jnp.float32),
            pltpu.SemaphoreType.DMA,
            pltpu.SemaphoreType.DMA,
            pltpu.SemaphoreType.DMA,
            pltpu.SemaphoreType.DMA,
            pltpu.SemaphoreType.DMA,
            pltpu.SemaphoreType.DMA,
        ],
    )


def _tc_x_body(xraw_ref, g_ref, b_ref, w1a_ref, w1b_ref,
               x0_ref, y1_ref, y2_ref):
    xr = xraw_ref[...]
    mu = jnp.mean(xr, axis=0, keepdims=True)
    xc = xr - mu
    var = jnp.mean(xc * xc, axis=0, keepdims=True)
    s = g_ref[...] * lax.rsqrt(var + _EPS)
    x0 = jnp.maximum(xc * s + b_ref[...], 0.0)
    x0_ref[...] = x0
    y1_ref[...] = jnp.dot(x0, w1a_ref[...], preferred_element_type=jnp.float32)
    y2_ref[...] = jnp.dot(x0, w1b_ref[...], preferred_element_type=jnp.float32)


def _make_tc_x():
    spec = pl.BlockSpec((_N, _F), lambda: (0, 0))
    vec = pl.BlockSpec((1, _F), lambda: (0, 0))
    w = pl.BlockSpec((_F, _F), lambda: (0, 0))
    return pl.pallas_call(
        _tc_x_body,
        in_specs=[spec, vec, vec, w, w],
        out_specs=[spec, spec, spec],
        out_shape=[jax.ShapeDtypeStruct((_N, _F), jnp.float32)] * 3,
    )


_BT = 4096
_NBT = _E // _BT
_ZT = 1024
_NZ = _E // _ZT


def _tc_bz_body(fuse_y, d3_ref, a_ref, sa_ref, g1_ref, b1_ref, w2t_ref,
                g2_ref, b2_ref, w1a_ref, w1b_ref, z_ref, y1_ref, y2_ref,
                bscr_ref, acc_ref, zacc_ref):
    i = pl.program_id(0)

    @pl.when(i == 0)
    def _():
        acc_ref[...] = jnp.zeros_like(acc_ref)
        zacc_ref[...] = jnp.zeros_like(zacc_ref)

    @pl.when(i < _NBT)
    def _():
        sa = sa_ref[...]
        mu = jnp.sum(sa[: _NW], axis=0, keepdims=True) * (1.0 / _E)
        var = jnp.sum(sa[_NW:], axis=0, keepdims=True) * (1.0 / _E) - mu * mu
        s = g1_ref[...] * lax.rsqrt(var + _EPS)
        t = b1_ref[...] - mu * s
        act = jnp.maximum(a_ref[...] * s + t, 0.0)
        bt = jnp.dot(act.astype(jnp.bfloat16),
                     w2t_ref[...].astype(jnp.bfloat16),
                     preferred_element_type=jnp.float32)
        bscr_ref[pl.ds(i * _BT, _BT), :] = bt
        acc_ref[0:1] = acc_ref[0:1] + jnp.sum(bt, axis=0, keepdims=True)
        acc_ref[1:2] = acc_ref[1:2] + jnp.sum(bt * bt, axis=0, keepdims=True)

    @pl.when(i >= _NBT)
    def _():
        j = i - _NBT
        mu = acc_ref[0:1] * (1.0 / _E)
        var = acc_ref[1:2] * (1.0 / _E) - mu * mu
        s = g2_ref[...] * lax.rsqrt(var + _EPS)
        t = b2_ref[...] - mu * s
        h = jnp.maximum(bscr_ref[pl.ds(j * _ZT, _ZT), :] * s + t, 0.0)
        zacc_ref[...] = zacc_ref[...] + jnp.dot(
            d3_ref[...].astype(jnp.bfloat16), h.astype(jnp.bfloat16),
            preferred_element_type=jnp.float32)

    @pl.when(i == _NBT + _NZ - 1)
    def _():
        z = zacc_ref[...]
        z_ref[...] = z
        if fuse_y:
            y1_ref[...] = jnp.dot(z, w1a_ref[...],
                                  preferred_element_type=jnp.float32)
            y2_ref[...] = jnp.dot(z, w1b_ref[...],
                                  preferred_element_type=jnp.float32)


def _make_tc_bz(fuse_y):
    d3_spec = pl.BlockSpec(
        (_N, _ZT), lambda i: (0, jnp.clip(i - _NBT, 0, _NZ - 1)))
    a_spec = pl.BlockSpec((_BT, _F), lambda i: (jnp.minimum(i, _NBT - 1), 0))
    sa_spec = pl.BlockSpec((2 * _NW, _F), lambda i: (0, 0))
    vec = pl.BlockSpec((1, _F), lambda i: (0, 0))
    w = pl.BlockSpec((_F, _F), lambda i: (0, 0))
    full = pl.BlockSpec((_N, _F), lambda i: (0, 0))
    n_out = 3 if fuse_y else 1
    if fuse_y:
        body = functools.partial(_tc_bz_body, True)
        in_specs = [d3_spec, a_spec, sa_spec, vec, vec, w, vec, vec, w, w]
    else:
        def body(d3_ref, a_ref, sa_ref, g1_ref, b1_ref, w2t_ref,
                 g2_ref, b2_ref, z_ref, bscr_ref, acc_ref, zacc_ref):
            return _tc_bz_body(False, d3_ref, a_ref, sa_ref, g1_ref, b1_ref,
                               w2t_ref, g2_ref, b2_ref, None, None,
                               z_ref, None, None, bscr_ref, acc_ref, zacc_ref)
        in_specs = [d3_spec, a_spec, sa_spec, vec, vec, w, vec, vec]
    return pl.pallas_call(
        body,
        grid=(_NBT + _NZ,),
        in_specs=in_specs,
        out_specs=[full] * n_out,
        out_shape=[jax.ShapeDtypeStruct((_N, _F), jnp.float32)] * n_out,
        scratch_shapes=[
            pltpu.VMEM((_E, _F), jnp.float32),
            pltpu.VMEM((8, _F), jnp.float32),
            pltpu.VMEM((_N, _F), jnp.float32),
        ],
    )


def _sc_emb(table, idx):
    return _make_sc_emb()(table, idx)


def _sc_edge(y1, y2, i1, i2):
    return _make_sc_edge()(y1, y2, i1, i2)


_tc_x = _make_tc_x()
_tc_bz_fused = _make_tc_bz(True)
_tc_bz_last = _make_tc_bz(False)


def kernel(data0, data1, data2, data3, emb, bn0_g, bn0_b,
           W1, bn1_g, bn1_b, W2, bn2_g, bn2_b):
    d0 = data0.astype(jnp.int32)
    d1 = data1.astype(jnp.int32)
    d2 = data2.astype(jnp.int32)

    xraw = _sc_emb(emb, d0)
    x0, y1, y2 = _tc_x(xraw, bn0_g.reshape(1, _F), bn0_b.reshape(1, _F),
                       W1[0, :, :_F].T, W1[0, :, _F:].T)

    outs = [x0]
    for i in range(2):
        a, sa = _sc_edge(y1, y2, d1, d2)
        g1 = bn1_g[i].reshape(1, _F)
        b1 = bn1_b[i].reshape(1, _F)
        g2 = bn2_g[i].reshape(1, _F)
        b2 = bn2_b[i].reshape(1, _F)
        if i == 0:
            z, y1, y2 = _tc_bz_fused(data3, a, sa, g1, b1, W2[i].T, g2, b2,
                                     W1[1, :, :_F].T, W1[1, :, _F:].T)
        else:
            (z,) = _tc_bz_last(data3, a, sa, g1, b1, W2[i].T, g2, b2)
        outs.append(z)
    return tuple(outs)


# speedup vs baseline: 1.0422x; 1.0422x over previous
import functools

import jax
import jax.numpy as jnp
from jax import lax
from jax.experimental import pallas as pl
from jax.experimental.pallas import tpu as pltpu
from jax.experimental.pallas import tpu_sc as plsc

_N = 2048
_E = 32768
_F = 128
_EPS = 1e-5

_NC = 2
_NS = 16
_NW = _NC * _NS

_SC_MESH = dict(core_axis_name="c", subcore_axis_name="s", num_cores=_NC,
                num_subcores=_NS)


def _sc_emb_body(table, idx, out, idx_v, rows_v, sem):
    wid = lax.axis_index("s") * _NC + lax.axis_index("c")
    rows = _N // _NW
    base = wid * rows
    pltpu.sync_copy(idx.at[pl.ds(base, rows)], idx_v)
    pltpu.async_copy(table.at[idx_v], rows_v, sem).wait()
    pltpu.sync_copy(rows_v, out.at[pl.ds(base, rows)])


@functools.cache
def _make_sc_emb():
    rows = _N // _NW
    return pl.kernel(
        _sc_emb_body,
        out_type=jax.ShapeDtypeStruct((_N, _F), jnp.float32),
        mesh=plsc.VectorSubcoreMesh(**_SC_MESH),
        scratch_types=[
            pltpu.VMEM((rows,), jnp.int32),
            pltpu.VMEM((rows, _F), jnp.float32),
            pltpu.SemaphoreType.DMA,
        ],
    )


_EPW = _E // _NW
_CH = 128
_NCH = _EPW // _CH


def _sc_edge_body(y1, y2, i1, i2, out, stats,
                  i1_v, i2_v, r1_0, r2_0, r1_1, r2_1, o_0, o_1, sv0, sv1,
                  sg1_0, sg2_0, sg1_1, sg2_1, so_0, so_1):
    wid = lax.axis_index("s") * _NC + lax.axis_index("c")
    base = wid * _EPW
    pltpu.sync_copy(i1.at[pl.ds(base, _EPW)], i1_v)
    pltpu.sync_copy(i2.at[pl.ds(base, _EPW)], i2_v)
    r1 = (r1_0, r1_1)
    r2 = (r2_0, r2_1)
    o = (o_0, o_1)
    sg1 = (sg1_0, sg1_1)
    sg2 = (sg2_0, sg2_1)
    so = (so_0, so_1)

    def fire(c):
        b = c & 1
        cp1 = pltpu.async_copy(y1.at[i1_v.at[pl.ds(c * _CH, _CH)]],
                               r1[b], sg1[b])
        cp2 = pltpu.async_copy(y2.at[i2_v.at[pl.ds(c * _CH, _CH)]],
                               r2[b], sg2[b])
        return cp1, cp2

    pending = {0: fire(0)}
    out_pending = {}
    carry = tuple(jnp.zeros((16,), jnp.float32) for _ in range(16))
    for c in range(_NCH):
        b = c & 1
        if c + 1 < _NCH:
            pending[c + 1] = fire(c + 1)
        cp1, cp2 = pending.pop(c)
        cp1.wait()
        cp2.wait()
        if c >= 2:
            out_pending.pop(c - 2).wait()
        r1v, r2v, ov = r1[b], r2[b], o[b]

        def row_body(r, carry):
            new = list(carry)
            for j in range(_F // 16):
                sl = pl.ds(j * 16, 16)
                a = r1v[r, sl] + r2v[r, sl]
                ov[r, sl] = a
                new[j] = new[j] + a
                new[8 + j] = new[8 + j] + a * a
            return tuple(new)

        carry = lax.fori_loop(0, _CH, row_body, carry)
        out_pending[c] = pltpu.async_copy(
            ov, out.at[pl.ds(base + c * _CH, _CH)], so[b])
    for c in sorted(out_pending):
        out_pending.pop(c).wait()
    for j in range(_F // 16):
        sl = pl.ds(j * 16, 16)
        sv0[0, sl] = carry[j]
        sv1[0, sl] = carry[8 + j]
    pltpu.sync_copy(sv0, stats.at[pl.ds(wid, 1)])
    pltpu.sync_copy(sv1, stats.at[pl.ds(_NW + wid, 1)])


@functools.cache
def _make_sc_edge():
    return pl.kernel(
        _sc_edge_body,
        out_type=[
            jax.ShapeDtypeStruct((_E, _F), jnp.float32),
            jax.ShapeDtypeStruct((2 * _NW, _F), jnp.float32),
        ],
        mesh=plsc.VectorSubcoreMesh(**_SC_MESH),
        scratch_types=[
            pltpu.VMEM((_EPW,), jnp.int32),
            pltpu.VMEM((_EPW,), jnp.int32),
            pltpu.VMEM((_CH, _F), jnp.float32),
            pltpu.VMEM((_CH, _F), jnp.float32),
            pltpu.VMEM((_CH, _F), jnp.float32),
            pltpu.VMEM((_CH, _F), jnp.float32),
            pltpu.VMEM((_CH, _F), jnp.float32),
            pltpu.VMEM((_CH, _F), jnp.float32),
            pltpu.VMEM((1, _F), jnp.float32),
            pltpu.VMEM((1, _F), jnp.float32),
            pltpu.SemaphoreType.DMA,
            pltpu.SemaphoreType.DMA,
            pltpu.SemaphoreType.DMA,
            pltpu.SemaphoreType.DMA,
            pltpu.SemaphoreType.DMA,
            pltpu.SemaphoreType.DMA,
        ],
    )


def _tc_x_body(xraw_ref, g_ref, b_ref, w1a_ref, w1b_ref,
               x0_ref, y1_ref, y2_ref):
    xr = xraw_ref[...]
    mu = jnp.mean(xr, axis=0, keepdims=True)
    xc = xr - mu
    var = jnp.mean(xc * xc, axis=0, keepdims=True)
    s = g_ref[...] * lax.rsqrt(var + _EPS)
    x0 = jnp.maximum(xc * s + b_ref[...], 0.0)
    x0_ref[...] = x0
    y1_ref[...] = jnp.dot(x0, w1a_ref[...], preferred_element_type=jnp.float32)
    y2_ref[...] = jnp.dot(x0, w1b_ref[...], preferred_element_type=jnp.float32)


def _make_tc_x():
    spec = pl.BlockSpec((_N, _F), lambda: (0, 0))
    vec = pl.BlockSpec((1, _F), lambda: (0, 0))
    w = pl.BlockSpec((_F, _F), lambda: (0, 0))
    return pl.pallas_call(
        _tc_x_body,
        in_specs=[spec, vec, vec, w, w],
        out_specs=[spec, spec, spec],
        out_shape=[jax.ShapeDtypeStruct((_N, _F), jnp.float32)] * 3,
    )


_BT = 4096
_NBT = _E // _BT
_ZRB = 512
_ZCB = 4096
_NRBZ = _N // _ZRB
_NCBZ = _E // _ZCB
_NZ = _NRBZ * _NCBZ


def _tc_bz_body(fuse_y, d3_ref, a_ref, sa_ref, g1_ref, b1_ref, w2t_ref,
                g2_ref, b2_ref, w1a_ref, w1b_ref, z_ref, y1_ref, y2_ref,
                bscr_ref, hscr_ref, acc_ref, zacc_ref):
    i = pl.program_id(0)

    @pl.when(i == 0)
    def _():
        acc_ref[...] = jnp.zeros_like(acc_ref)
        zacc_ref[...] = jnp.zeros_like(zacc_ref)

    @pl.when(i < _NBT)
    def _():
        sa = sa_ref[...]
        mu = jnp.sum(sa[: _NW], axis=0, keepdims=True) * (1.0 / _E)
        var = jnp.sum(sa[_NW:], axis=0, keepdims=True) * (1.0 / _E) - mu * mu
        s = g1_ref[...] * lax.rsqrt(var + _EPS)
        t = b1_ref[...] - mu * s
        act = jnp.maximum(a_ref[...] * s + t, 0.0)
        bt = jnp.dot(act.astype(jnp.bfloat16),
                     w2t_ref[...].astype(jnp.bfloat16),
                     preferred_element_type=jnp.float32)
        bscr_ref[pl.ds(i * _BT, _BT), :] = bt
        acc_ref[0:1] = acc_ref[0:1] + jnp.sum(bt, axis=0, keepdims=True)
        acc_ref[1:2] = acc_ref[1:2] + jnp.sum(bt * bt, axis=0, keepdims=True)

    @pl.when(i >= _NBT)
    def _():
        j = i - _NBT
        rb = lax.div(j, _NCBZ)
        cb = lax.rem(j, _NCBZ)
        csl = pl.ds(cb * _ZCB, _ZCB)

        @pl.when(rb == 0)
        def _():
            mu = acc_ref[0:1] * (1.0 / _E)
            var = acc_ref[1:2] * (1.0 / _E) - mu * mu
            s = g2_ref[...] * lax.rsqrt(var + _EPS)
            t = b2_ref[...] - mu * s
            hscr_ref[csl, :] = jnp.maximum(
                bscr_ref[csl, :] * s + t, 0.0).astype(jnp.bfloat16)

        rsl = pl.ds(rb * _ZRB, _ZRB)
        zacc_ref[rsl, :] = zacc_ref[rsl, :] + jnp.dot(
            d3_ref[...].astype(jnp.bfloat16), hscr_ref[csl, :],
            preferred_element_type=jnp.float32)

    @pl.when(i == _NBT + _NZ - 1)
    def _():
        z = zacc_ref[...]
        z_ref[...] = z
        if fuse_y:
            y1_ref[...] = jnp.dot(z, w1a_ref[...],
                                  preferred_element_type=jnp.float32)
            y2_ref[...] = jnp.dot(z, w1b_ref[...],
                                  preferred_element_type=jnp.float32)


def _make_tc_bz(fuse_y):
    def d3_map(i):
        j = jnp.clip(i - _NBT, 0, _NZ - 1)
        return (lax.div(j, _NCBZ), lax.rem(j, _NCBZ))

    d3_spec = pl.BlockSpec((_ZRB, _ZCB), d3_map)
    a_spec = pl.BlockSpec((_BT, _F), lambda i: (jnp.minimum(i, _NBT - 1), 0))
    sa_spec = pl.BlockSpec((2 * _NW, _F), lambda i: (0, 0))
    vec = pl.BlockSpec((1, _F), lambda i: (0, 0))
    w = pl.BlockSpec((_F, _F), lambda i: (0, 0))
    full = pl.BlockSpec((_N, _F), lambda i: (0, 0))
    n_out = 3 if fuse_y else 1
    if fuse_y:
        body = functools.partial(_tc_bz_body, True)
        in_specs = [d3_spec, a_spec, sa_spec, vec, vec, w, vec, vec, w, w]
    else:
        def body(d3_ref, a_ref, sa_ref, g1_ref, b1_ref, w2t_ref,
                 g2_ref, b2_ref, z_ref, bscr_ref, hscr_ref, acc_ref,
                 zacc_ref):
            return _tc_bz_body(False, d3_ref, a_ref, sa_ref, g1_ref, b1_ref,
                               w2t_ref, g2_ref, b2_ref, None, None,
                               z_ref, None, None, bscr_ref, hscr_ref,
                               acc_ref, zacc_ref)
        in_specs = [d3_spec, a_spec, sa_spec, vec, vec, w, vec, vec]
    return pl.pallas_call(
        body,
        grid=(_NBT + _NZ,),
        in_specs=in_specs,
        out_specs=[full] * n_out,
        out_shape=[jax.ShapeDtypeStruct((_N, _F), jnp.float32)] * n_out,
        scratch_shapes=[
            pltpu.VMEM((_E, _F), jnp.float32),
            pltpu.VMEM((_E, _F), jnp.bfloat16),
            pltpu.VMEM((8, _F), jnp.float32),
            pltpu.VMEM((_N, _F), jnp.float32),
        ],
    )


def _sc_emb(table, idx):
    return _make_sc_emb()(table, idx)


def _sc_edge(y1, y2, i1, i2):
    return _make_sc_edge()(y1, y2, i1, i2)


_tc_x = _make_tc_x()
_tc_bz_fused = _make_tc_bz(True)
_tc_bz_last = _make_tc_bz(False)


def kernel(data0, data1, data2, data3, emb, bn0_g, bn0_b,
           W1, bn1_g, bn1_b, W2, bn2_g, bn2_b):
    d0 = data0.astype(jnp.int32)
    d1 = data1.astype(jnp.int32)
    d2 = data2.astype(jnp.int32)

    xraw = _sc_emb(emb, d0)
    x0, y1, y2 = _tc_x(xraw, bn0_g.reshape(1, _F), bn0_b.reshape(1, _F),
                       W1[0, :, :_F].T, W1[0, :, _F:].T)

    outs = [x0]
    for i in range(2):
        a, sa = _sc_edge(y1, y2, d1, d2)
        g1 = bn1_g[i].reshape(1, _F)
        b1 = bn1_b[i].reshape(1, _F)
        g2 = bn2_g[i].reshape(1, _F)
        b2 = bn2_b[i].reshape(1, _F)
        if i == 0:
            z, y1, y2 = _tc_bz_fused(data3, a, sa, g1, b1, W2[i].T, g2, b2,
                                     W1[1, :, :_F].T, W1[1, :, _F:].T)
        else:
            (z,) = _tc_bz_last(data3, a, sa, g1, b1, W2[i].T, g2, b2)
        outs.append(z)
    return tuple(outs)


# speedup vs baseline: 1.0479x; 1.0055x over previous
import functools

import jax
import jax.numpy as jnp
from jax import lax
from jax.experimental import pallas as pl
from jax.experimental.pallas import tpu as pltpu
from jax.experimental.pallas import tpu_sc as plsc

_N = 2048
_E = 32768
_F = 128
_EPS = 1e-5

_NC = 2
_NS = 16
_NW = _NC * _NS

_SC_MESH = dict(core_axis_name="c", subcore_axis_name="s", num_cores=_NC,
                num_subcores=_NS)


def _sc_emb_body(table, idx, out, idx_v, rows_v, sem):
    wid = lax.axis_index("s") * _NC + lax.axis_index("c")
    rows = _N // _NW
    base = wid * rows
    pltpu.sync_copy(idx.at[pl.ds(base, rows)], idx_v)
    pltpu.async_copy(table.at[idx_v], rows_v, sem).wait()
    pltpu.sync_copy(rows_v, out.at[pl.ds(base, rows)])


@functools.cache
def _make_sc_emb():
    rows = _N // _NW
    return pl.kernel(
        _sc_emb_body,
        out_type=jax.ShapeDtypeStruct((_N, _F), jnp.float32),
        mesh=plsc.VectorSubcoreMesh(**_SC_MESH),
        scratch_types=[
            pltpu.VMEM((rows,), jnp.int32),
            pltpu.VMEM((rows, _F), jnp.float32),
            pltpu.SemaphoreType.DMA,
        ],
    )


_EPW = _E // _NW
_CH = 128
_NCH = _EPW // _CH


def _sc_edge_body(y1, y2, i1, i2, out, stats,
                  i1_v, i2_v, r1_0, r2_0, r1_1, r2_1, o_0, o_1, sv0, sv1,
                  sg1_0, sg2_0, sg1_1, sg2_1, so_0, so_1):
    wid = lax.axis_index("s") * _NC + lax.axis_index("c")
    base = wid * _EPW
    pltpu.sync_copy(i1.at[pl.ds(base, _EPW)], i1_v)
    pltpu.sync_copy(i2.at[pl.ds(base, _EPW)], i2_v)
    r1 = (r1_0, r1_1)
    r2 = (r2_0, r2_1)
    o = (o_0, o_1)
    sg1 = (sg1_0, sg1_1)
    sg2 = (sg2_0, sg2_1)
    so = (so_0, so_1)

    def fire(c):
        b = c & 1
        cp1 = pltpu.async_copy(y1.at[i1_v.at[pl.ds(c * _CH, _CH)]],
                               r1[b], sg1[b])
        cp2 = pltpu.async_copy(y2.at[i2_v.at[pl.ds(c * _CH, _CH)]],
                               r2[b], sg2[b])
        return cp1, cp2

    pending = {0: fire(0)}
    out_pending = {}
    carry = tuple(jnp.zeros((16,), jnp.float32) for _ in range(16))
    for c in range(_NCH):
        b = c & 1
        if c + 1 < _NCH:
            pending[c + 1] = fire(c + 1)
        cp1, cp2 = pending.pop(c)
        cp1.wait()
        cp2.wait()
        if c >= 2:
            out_pending.pop(c - 2).wait()
        r1v, r2v, ov = r1[b], r2[b], o[b]

        def row_body(r, carry):
            new = list(carry)
            for j in range(_F // 16):
                sl = pl.ds(j * 16, 16)
                a = r1v[r, sl] + r2v[r, sl]
                ov[r, sl] = a
                new[j] = new[j] + a
                new[8 + j] = new[8 + j] + a * a
            return tuple(new)

        carry = lax.fori_loop(0, _CH, row_body, carry)
        out_pending[c] = pltpu.async_copy(
            ov, out.at[pl.ds(base + c * _CH, _CH)], so[b])
    for c in sorted(out_pending):
        out_pending.pop(c).wait()
    for j in range(_F // 16):
        sl = pl.ds(j * 16, 16)
        sv0[0, sl] = carry[j]
        sv1[0, sl] = carry[8 + j]
    pltpu.sync_copy(sv0, stats.at[pl.ds(wid, 1)])
    pltpu.sync_copy(sv1, stats.at[pl.ds(_NW + wid, 1)])


@functools.cache
def _make_sc_edge():
    return pl.kernel(
        _sc_edge_body,
        out_type=[
            jax.ShapeDtypeStruct((_E, _F), jnp.float32),
            jax.ShapeDtypeStruct((2 * _NW, _F), jnp.float32),
        ],
        mesh=plsc.VectorSubcoreMesh(**_SC_MESH),
        scratch_types=[
            pltpu.VMEM((_EPW,), jnp.int32),
            pltpu.VMEM((_EPW,), jnp.int32),
            pltpu.VMEM((_CH, _F), jnp.float32),
            pltpu.VMEM((_CH, _F), jnp.float32),
            pltpu.VMEM((_CH, _F), jnp.float32),
            pltpu.VMEM((_CH, _F), jnp.float32),
            pltpu.VMEM((_CH, _F), jnp.float32),
            pltpu.VMEM((_CH, _F), jnp.float32),
            pltpu.VMEM((1, _F), jnp.float32),
            pltpu.VMEM((1, _F), jnp.float32),
            pltpu.SemaphoreType.DMA,
            pltpu.SemaphoreType.DMA,
            pltpu.SemaphoreType.DMA,
            pltpu.SemaphoreType.DMA,
            pltpu.SemaphoreType.DMA,
            pltpu.SemaphoreType.DMA,
        ],
    )


def _tc_x_body(xraw_ref, g_ref, b_ref, w1a_ref, w1b_ref,
               x0_ref, y1_ref, y2_ref):
    xr = xraw_ref[...]
    mu = jnp.mean(xr, axis=0, keepdims=True)
    xc = xr - mu
    var = jnp.mean(xc * xc, axis=0, keepdims=True)
    s = g_ref[...] * lax.rsqrt(var + _EPS)
    x0 = jnp.maximum(xc * s + b_ref[...], 0.0)
    x0_ref[...] = x0
    y1_ref[...] = jnp.dot(x0, w1a_ref[...], preferred_element_type=jnp.float32)
    y2_ref[...] = jnp.dot(x0, w1b_ref[...], preferred_element_type=jnp.float32)


def _make_tc_x():
    spec = pl.BlockSpec((_N, _F), lambda: (0, 0))
    vec = pl.BlockSpec((1, _F), lambda: (0, 0))
    w = pl.BlockSpec((_F, _F), lambda: (0, 0))
    return pl.pallas_call(
        _tc_x_body,
        in_specs=[spec, vec, vec, w, w],
        out_specs=[spec, spec, spec],
        out_shape=[jax.ShapeDtypeStruct((_N, _F), jnp.float32)] * 3,
    )


_BT = 4096
_NBT = _E // _BT
_ZRB = 256
_ZCB = 8192
_NRBZ = _N // _ZRB
_NCBZ = _E // _ZCB
_NZ = _NRBZ * _NCBZ


def _tc_bz_body(fuse_y, d3_ref, a_ref, sa_ref, g1_ref, b1_ref, w2t_ref,
                g2_ref, b2_ref, w1a_ref, w1b_ref, z_ref, y1_ref, y2_ref,
                bscr_ref, hscr_ref, acc_ref, zacc_ref):
    i = pl.program_id(0)

    @pl.when(i == 0)
    def _():
        acc_ref[...] = jnp.zeros_like(acc_ref)
        zacc_ref[...] = jnp.zeros_like(zacc_ref)

    @pl.when(i < _NBT)
    def _():
        sa = sa_ref[...]
        mu = jnp.sum(sa[: _NW], axis=0, keepdims=True) * (1.0 / _E)
        var = jnp.sum(sa[_NW:], axis=0, keepdims=True) * (1.0 / _E) - mu * mu
        s = g1_ref[...] * lax.rsqrt(var + _EPS)
        t = b1_ref[...] - mu * s
        act = jnp.maximum(a_ref[...] * s + t, 0.0)
        bt = jnp.dot(act.astype(jnp.bfloat16),
                     w2t_ref[...].astype(jnp.bfloat16),
                     preferred_element_type=jnp.float32)
        bscr_ref[pl.ds(i * _BT, _BT), :] = bt
        acc_ref[0:1] = acc_ref[0:1] + jnp.sum(bt, axis=0, keepdims=True)
        acc_ref[1:2] = acc_ref[1:2] + jnp.sum(bt * bt, axis=0, keepdims=True)

    @pl.when(i >= _NBT)
    def _():
        j = i - _NBT
        rb = lax.div(j, _NCBZ)
        cb = lax.rem(j, _NCBZ)
        csl = pl.ds(cb * _ZCB, _ZCB)

        @pl.when(rb == 0)
        def _():
            mu = acc_ref[0:1] * (1.0 / _E)
            var = acc_ref[1:2] * (1.0 / _E) - mu * mu
            s = g2_ref[...] * lax.rsqrt(var + _EPS)
            t = b2_ref[...] - mu * s
            hscr_ref[csl, :] = jnp.maximum(
                bscr_ref[csl, :] * s + t, 0.0).astype(jnp.bfloat16)

        rsl = pl.ds(rb * _ZRB, _ZRB)
        zacc_ref[rsl, :] = zacc_ref[rsl, :] + jnp.dot(
            d3_ref[...].astype(jnp.bfloat16), hscr_ref[csl, :],
            preferred_element_type=jnp.float32)

    @pl.when(i == _NBT + _NZ - 1)
    def _():
        z = zacc_ref[...]
        z_ref[...] = z
        if fuse_y:
            y1_ref[...] = jnp.dot(z, w1a_ref[...],
                                  preferred_element_type=jnp.float32)
            y2_ref[...] = jnp.dot(z, w1b_ref[...],
                                  preferred_element_type=jnp.float32)


def _make_tc_bz(fuse_y):
    def d3_map(i):
        j = jnp.clip(i - _NBT, 0, _NZ - 1)
        return (lax.div(j, _NCBZ), lax.rem(j, _NCBZ))

    d3_spec = pl.BlockSpec((_ZRB, _ZCB), d3_map)
    a_spec = pl.BlockSpec((_BT, _F), lambda i: (jnp.minimum(i, _NBT - 1), 0))
    sa_spec = pl.BlockSpec((2 * _NW, _F), lambda i: (0, 0))
    vec = pl.BlockSpec((1, _F), lambda i: (0, 0))
    w = pl.BlockSpec((_F, _F), lambda i: (0, 0))
    full = pl.BlockSpec((_N, _F), lambda i: (0, 0))
    n_out = 3 if fuse_y else 1
    if fuse_y:
        body = functools.partial(_tc_bz_body, True)
        in_specs = [d3_spec, a_spec, sa_spec, vec, vec, w, vec, vec, w, w]
    else:
        def body(d3_ref, a_ref, sa_ref, g1_ref, b1_ref, w2t_ref,
                 g2_ref, b2_ref, z_ref, bscr_ref, hscr_ref, acc_ref,
                 zacc_ref):
            return _tc_bz_body(False, d3_ref, a_ref, sa_ref, g1_ref, b1_ref,
                               w2t_ref, g2_ref, b2_ref, None, None,
                               z_ref, None, None, bscr_ref, hscr_ref,
                               acc_ref, zacc_ref)
        in_specs = [d3_spec, a_spec, sa_spec, vec, vec, w, vec, vec]
    return pl.pallas_call(
        body,
        grid=(_NBT + _NZ,),
        in_specs=in_specs,
        out_specs=[full] * n_out,
        out_shape=[jax.ShapeDtypeStruct((_N, _F), jnp.float32)] * n_out,
        scratch_shapes=[
            pltpu.VMEM((_E, _F), jnp.float32),
            pltpu.VMEM((_E, _F), jnp.bfloat16),
            pltpu.VMEM((8, _F), jnp.float32),
            pltpu.VMEM((_N, _F), jnp.float32),
        ],
    )


def _sc_emb(table, idx):
    return _make_sc_emb()(table, idx)


def _sc_edge(y1, y2, i1, i2):
    return _make_sc_edge()(y1, y2, i1, i2)


_tc_x = _make_tc_x()
_tc_bz_fused = _make_tc_bz(True)
_tc_bz_last = _make_tc_bz(False)


def kernel(data0, data1, data2, data3, emb, bn0_g, bn0_b,
           W1, bn1_g, bn1_b, W2, bn2_g, bn2_b):
    d0 = data0.astype(jnp.int32)
    d1 = data1.astype(jnp.int32)
    d2 = data2.astype(jnp.int32)

    xraw = _sc_emb(emb, d0)
    x0, y1, y2 = _tc_x(xraw, bn0_g.reshape(1, _F), bn0_b.reshape(1, _F),
                       W1[0, :, :_F].T, W1[0, :, _F:].T)

    outs = [x0]
    for i in range(2):
        a, sa = _sc_edge(y1, y2, d1, d2)
        g1 = bn1_g[i].reshape(1, _F)
        b1 = bn1_b[i].reshape(1, _F)
        g2 = bn2_g[i].reshape(1, _F)
        b2 = bn2_b[i].reshape(1, _F)
        if i == 0:
            z, y1, y2 = _tc_bz_fused(data3, a, sa, g1, b1, W2[i].T, g2, b2,
                                     W1[1, :, :_F].T, W1[1, :, _F:].T)
        else:
            (z,) = _tc_bz_last(data3, a, sa, g1, b1, W2[i].T, g2, b2)
        outs.append(z)
    return tuple(outs)
